# Initial kernel scaffold; baseline (speedup 1.0000x reference)
import jax
import jax.numpy as jnp
from jax.experimental import pallas as pl


def kernel(x, table):
    raise NotImplementedError("write your pallas kernel here")


# speedup vs baseline: 715.9775x; 715.9775x over previous
import functools

import jax
import jax.numpy as jnp
from jax import lax
from jax.experimental import pallas as pl
from jax.experimental.pallas import tpu as pltpu
from jax.experimental.pallas import tpu_sc as plsc

_NC = 2
_NS = 16
_NW = _NC * _NS
_L = 16


@functools.lru_cache(maxsize=None)
def _make_sc_broadcast(B, V):
    rows_w = B // _NW
    R = 64
    n_dma = rows_w // R
    n_chunk = V // _L
    rem = V - n_chunk * _L

    mesh = plsc.VectorSubcoreMesh(core_axis_name="c", subcore_axis_name="s")

    @functools.partial(
        pl.kernel,
        out_type=jax.ShapeDtypeStruct((B, V), jnp.float32),
        mesh=mesh,
        scratch_types=[
            pltpu.VMEM((V,), jnp.float32),
            pltpu.VMEM((R, V), jnp.float32),
            pltpu.SemaphoreType.DMA,
        ],
    )
    def broadcast_kernel(table_hbm, out_hbm, tab_v, buf_v, sem):
        wid = lax.axis_index("s") * _NC + lax.axis_index("c")
        base = wid * rows_w
        del tab_v
        fills = [
            pltpu.async_copy(table_hbm, buf_v.at[r], sem) for r in range(R)
        ]
        for cp in fills:
            cp.wait()

        copies = [
            pltpu.async_copy(buf_v, out_hbm.at[pl.ds(base + c * R, R)], sem)
            for c in range(n_dma)
        ]
        for cp in copies:
            cp.wait()

    return broadcast_kernel


def kernel(x, table):
    B = x.shape[0]
    V = table.shape[0]
    fn = _make_sc_broadcast(B, V)
    return fn(table.reshape(V))


# speedup vs baseline: 787.8717x; 1.1004x over previous
import functools

import jax
import jax.numpy as jnp
from jax import lax
from jax.experimental import pallas as pl
from jax.experimental.pallas import tpu as pltpu
from jax.experimental.pallas import tpu_sc as plsc

_NC = 2
_NS = 16


@functools.lru_cache(maxsize=None)
def _make_sc_broadcast(B, V):
    rows_sc = B // _NC
    R = 512
    rows_fill = R // _NS
    n_dma = rows_sc // (_NS * R) * 1
    assert rows_sc % R == 0

    mesh = plsc.VectorSubcoreMesh(core_axis_name="c", subcore_axis_name="s")

    @functools.partial(
        pl.kernel,
        out_type=jax.ShapeDtypeStruct((B, V), jnp.float32),
        mesh=mesh,
        scratch_types=[
            pltpu.VMEM_SHARED((R, V), jnp.float32),
            pltpu.SemaphoreType.DMA,
        ],
    )
    def broadcast_kernel(table_hbm, out_hbm, shared_buf, sem):
        cid = lax.axis_index("c")
        sid = lax.axis_index("s")
        fills = [
            pltpu.async_copy(table_hbm, shared_buf.at[sid * rows_fill + r], sem)
            for r in range(rows_fill)
        ]
        for cp in fills:
            cp.wait()
        plsc.subcore_barrier()
        base = cid * rows_sc + sid * R
        n_out = rows_sc // (_NS * R)
        copies = [
            pltpu.async_copy(
                shared_buf,
                out_hbm.at[pl.ds(base + c * _NS * R, R)],
                sem,
            )
            for c in range(n_out)
        ]
        for cp in copies:
            cp.wait()

    return broadcast_kernel


def kernel(x, table):
    B = x.shape[0]
    V = table.shape[0]
    fn = _make_sc_broadcast(B, V)
    return fn(table.reshape(V))
